# baseline (speedup 1.0000x reference)
import functools

import jax
import jax.numpy as jnp
from jax import lax
from jax.experimental import pallas as pl
from jax.experimental.pallas import tpu as pltpu
from jax.experimental.pallas import tpu_sc as plsc

N_PAIRS = 16384
M_PAD = 1024
L = 16
NC, NS = 2, 16
NW = NC * NS
PAIRS_PER_W = N_PAIRS // NW
WORDS_PER_W = 2 * PAIRS_PER_W
CHUNKS = PAIRS_PER_W // L
AUX = M_PAD + L

_LN2 = 0.6931471805599453
_SQRT2 = 1.4142135


def _ln(y):
    yi = lax.bitcast_convert_type(y, jnp.int32)
    e = (yi >> 23) - 127
    m = lax.bitcast_convert_type((yi & 0x7FFFFF) | 0x3F800000, jnp.float32)
    big = m > _SQRT2
    m = jnp.where(big, m * 0.5, m)
    ef = (e + big.astype(jnp.int32)).astype(jnp.float32)
    t = (m - 1.0) / (m + 1.0)
    t2 = t * t
    p = 2.0 * t * (1.0 + t2 * (1.0 / 3.0 + t2 * (0.2 + t2 * (1.0 / 7.0))))
    return ef * _LN2 + p


def _body(pairs_hbm, aux_hbm, out_hbm, pairs_v, aux_v, acc_v):
    wid = lax.axis_index("s") * NC + lax.axis_index("c")
    row0 = wid * PAIRS_PER_W
    pltpu.sync_copy(pairs_hbm.at[pl.ds(row0, PAIRS_PER_W)], pairs_v)
    pltpu.sync_copy(aux_hbm, aux_v)

    b = aux_v[pl.ds(M_PAD, L)]
    lane = jnp.arange(L, dtype=jnp.int32)
    zeros = jnp.zeros((L,), jnp.int32)
    ones = zeros + 1

    def chunk(j, acc):
        rows = lane + (j * L)
        w_idx = plsc.load_gather(pairs_v, [rows, zeros])
        l_idx = plsc.load_gather(pairs_v, [rows, ones])
        uw = plsc.load_gather(aux_v, [w_idx])
        ul = plsc.load_gather(aux_v, [l_idx])
        y = 1.0 + jnp.exp(b * (ul - uw))
        return acc + _ln(y)

    acc_v[...] = lax.fori_loop(0, CHUNKS, chunk, jnp.zeros((L,), jnp.float32))
    pltpu.sync_copy(acc_v, out_hbm.at[wid])


_sc_call = pl.kernel(
    _body,
    out_type=jax.ShapeDtypeStruct((NW, L), jnp.float32),
    mesh=plsc.VectorSubcoreMesh(core_axis_name="c", subcore_axis_name="s"),
    compiler_params=pltpu.CompilerParams(
        needs_layout_passes=False, use_tc_tiling_on_sc=True),
    scratch_types=[
        pltpu.VMEM((PAIRS_PER_W, 2), jnp.int32),
        pltpu.VMEM((AUX,), jnp.float32),
        pltpu.VMEM((L,), jnp.float32),
    ],
)


def kernel(pairs, k, u, beta):
    b = jnp.where(k == 0, jnp.float32(1.0), beta[k]).astype(jnp.float32)
    aux = jnp.concatenate([
        u,
        jnp.zeros((M_PAD - u.shape[0],), jnp.float32),
        jnp.full((L,), b, jnp.float32),
    ])
    partials = _sc_call(pairs, aux)
    return jnp.sum(partials)


# speedup vs baseline: 1.3404x; 1.3404x over previous
import functools

import jax
import jax.numpy as jnp
from jax import lax
from jax.experimental import pallas as pl
from jax.experimental.pallas import tpu as pltpu
from jax.experimental.pallas import tpu_sc as plsc

N_PAIRS = 16384
M_PAD = 1024
L = 16
NC, NS = 2, 16
NW = NC * NS
PAIRS_PER_W = N_PAIRS // NW
WORDS_PER_W = 2 * PAIRS_PER_W
CHUNKS = PAIRS_PER_W // L
AUX = M_PAD + L

_LN2 = 0.6931471805599453
_SQRT2 = 1.4142135


def _ln(y):
    yi = lax.bitcast_convert_type(y, jnp.int32)
    e = (yi >> 23) - 127
    m = lax.bitcast_convert_type((yi & 0x7FFFFF) | 0x3F800000, jnp.float32)
    big = m > _SQRT2
    m = jnp.where(big, m * 0.5, m)
    ef = (e + big.astype(jnp.int32)).astype(jnp.float32)
    t = (m - 1.0) / (m + 1.0)
    t2 = t * t
    p = 2.0 * t * (1.0 + t2 * (1.0 / 3.0 + t2 * (0.2 + t2 * (1.0 / 7.0))))
    return ef * _LN2 + p


def _body(pairs_hbm, aux_hbm, out_hbm, pairs_v, aux_v, acc_v):
    wid = lax.axis_index("s") * NC + lax.axis_index("c")
    col0 = wid * PAIRS_PER_W
    pltpu.sync_copy(pairs_hbm.at[:, pl.ds(col0, PAIRS_PER_W)], pairs_v)
    pltpu.sync_copy(aux_hbm, aux_v)

    b = aux_v[pl.ds(M_PAD, L)]

    def chunk(j, acc):
        w_idx = pairs_v[0, pl.ds(j * L, L)]
        l_idx = pairs_v[1, pl.ds(j * L, L)]
        uw = plsc.load_gather(aux_v, [w_idx])
        ul = plsc.load_gather(aux_v, [l_idx])
        y = 1.0 + jnp.exp(b * (ul - uw))
        return acc + _ln(y)

    acc_v[...] = lax.fori_loop(0, CHUNKS, chunk, jnp.zeros((L,), jnp.float32))
    pltpu.sync_copy(acc_v, out_hbm.at[wid])


_sc_call = pl.kernel(
    _body,
    out_type=jax.ShapeDtypeStruct((NW, L), jnp.float32),
    mesh=plsc.VectorSubcoreMesh(core_axis_name="c", subcore_axis_name="s"),
    compiler_params=pltpu.CompilerParams(needs_layout_passes=False),
    scratch_types=[
        pltpu.VMEM((2, PAIRS_PER_W), jnp.int32),
        pltpu.VMEM((AUX,), jnp.float32),
        pltpu.VMEM((L,), jnp.float32),
    ],
)


def kernel(pairs, k, u, beta):
    b = jnp.where(k == 0, jnp.float32(1.0), beta[k]).astype(jnp.float32)
    aux = jnp.concatenate([
        u,
        jnp.zeros((M_PAD - u.shape[0],), jnp.float32),
        jnp.full((L,), b, jnp.float32),
    ])
    partials = _sc_call(pairs.T, aux)
    return jnp.sum(partials)


# speedup vs baseline: 1.4493x; 1.0813x over previous
import functools

import jax
import jax.numpy as jnp
from jax import lax
from jax.experimental import pallas as pl
from jax.experimental.pallas import tpu as pltpu
from jax.experimental.pallas import tpu_sc as plsc

N_PAIRS = 16384
M_PAD = 1024
L = 16
NC, NS = 1, 16
NW = NC * NS
PAIRS_PER_W = N_PAIRS // NW
WORDS_PER_W = 2 * PAIRS_PER_W
CHUNKS = PAIRS_PER_W // L
AUX = M_PAD + L

_LN2 = 0.6931471805599453
_SQRT2 = 1.4142135


def _ln(y):
    yi = lax.bitcast_convert_type(y, jnp.int32)
    e = (yi >> 23) - 127
    m = lax.bitcast_convert_type((yi & 0x7FFFFF) | 0x3F800000, jnp.float32)
    big = m > _SQRT2
    m = jnp.where(big, m * 0.5, m)
    ef = (e + big.astype(jnp.int32)).astype(jnp.float32)
    t = (m - 1.0) / (m + 1.0)
    t2 = t * t
    p = 2.0 * t * (1.0 + t2 * (1.0 / 3.0 + t2 * (0.2 + t2 * (1.0 / 7.0))))
    return ef * _LN2 + p


def _body(pairs_hbm, aux_hbm, out_hbm, pairs_v, aux_v, acc_v):
    wid = lax.axis_index("s") * NC + lax.axis_index("c")
    col0 = wid * PAIRS_PER_W
    pltpu.sync_copy(pairs_hbm.at[:, pl.ds(col0, PAIRS_PER_W)], pairs_v)
    pltpu.sync_copy(aux_hbm, aux_v)

    b = aux_v[pl.ds(M_PAD, L)]

    def chunk(j, acc):
        w_idx = pairs_v[0, pl.ds(j * L, L)]
        l_idx = pairs_v[1, pl.ds(j * L, L)]
        uw = plsc.load_gather(aux_v, [w_idx])
        ul = plsc.load_gather(aux_v, [l_idx])
        y = 1.0 + jnp.exp(b * (ul - uw))
        return acc + _ln(y)

    acc_v[...] = lax.fori_loop(0, CHUNKS, chunk, jnp.zeros((L,), jnp.float32))
    pltpu.sync_copy(acc_v, out_hbm.at[wid])


_sc_call = pl.kernel(
    _body,
    out_type=jax.ShapeDtypeStruct((NW, L), jnp.float32),
    mesh=plsc.VectorSubcoreMesh(core_axis_name="c", subcore_axis_name="s", num_cores=1, num_subcores=16),
    compiler_params=pltpu.CompilerParams(needs_layout_passes=False),
    scratch_types=[
        pltpu.VMEM((2, PAIRS_PER_W), jnp.int32),
        pltpu.VMEM((AUX,), jnp.float32),
        pltpu.VMEM((L,), jnp.float32),
    ],
)


def kernel(pairs, k, u, beta):
    b = jnp.where(k == 0, jnp.float32(1.0), beta[k]).astype(jnp.float32)
    aux = jnp.concatenate([
        u,
        jnp.zeros((M_PAD - u.shape[0],), jnp.float32),
        jnp.full((L,), b, jnp.float32),
    ])
    partials = _sc_call(pairs.T, aux)
    return jnp.sum(partials)


# speedup vs baseline: 1.4513x; 1.0014x over previous
import functools

import jax
import jax.numpy as jnp
from jax import lax
from jax.experimental import pallas as pl
from jax.experimental.pallas import tpu as pltpu
from jax.experimental.pallas import tpu_sc as plsc

N_PAIRS = 16384
M_PAD = 1024
L = 16
NC, NS = 1, 16
NW = NC * NS
PAIRS_PER_W = N_PAIRS // NW
WORDS_PER_W = 2 * PAIRS_PER_W
CHUNKS = PAIRS_PER_W // L
AUX = M_PAD + L

_LN2 = 0.6931471805599453
_SQRT2 = 1.4142135


def _ln(y):
    yi = lax.bitcast_convert_type(y, jnp.int32)
    e = (yi >> 23) - 127
    m = lax.bitcast_convert_type((yi & 0x7FFFFF) | 0x3F800000, jnp.float32)
    big = m > _SQRT2
    m = jnp.where(big, m * 0.5, m)
    ef = (e + big.astype(jnp.int32)).astype(jnp.float32)
    t = (m - 1.0) / (m + 1.0)
    t2 = t * t
    p = 2.0 * t * (1.0 + t2 * (1.0 / 3.0 + t2 * (0.2 + t2 * (1.0 / 7.0))))
    return ef * _LN2 + p


def _body(pairs_hbm, aux_hbm, out_hbm, pairs_v, aux_v, acc_v):
    wid = lax.axis_index("s") * NC + lax.axis_index("c")
    col0 = wid * PAIRS_PER_W
    pltpu.sync_copy(pairs_hbm.at[:, pl.ds(col0, PAIRS_PER_W)], pairs_v)
    pltpu.sync_copy(aux_hbm, aux_v)

    b = aux_v[pl.ds(M_PAD, L)]

    def chunk(j, acc):
        w_idx = pairs_v[0, pl.ds(j * L, L)]
        l_idx = pairs_v[1, pl.ds(j * L, L)]
        uw = plsc.load_gather(aux_v, [w_idx])
        ul = plsc.load_gather(aux_v, [l_idx])
        y = 1.0 + jnp.exp(b * (ul - uw))
        return acc + _ln(y)

    acc_v[...] = lax.fori_loop(0, CHUNKS, chunk, jnp.zeros((L,), jnp.float32))
    pltpu.sync_copy(acc_v, out_hbm.at[wid])


_sc_call = pl.kernel(
    _body,
    out_type=jax.ShapeDtypeStruct((NW, L), jnp.float32),
    mesh=plsc.VectorSubcoreMesh(core_axis_name="c", subcore_axis_name="s", num_cores=1, num_subcores=16),
    compiler_params=pltpu.CompilerParams(
        needs_layout_passes=False, skip_device_barrier=True),
    scratch_types=[
        pltpu.VMEM((2, PAIRS_PER_W), jnp.int32),
        pltpu.VMEM((AUX,), jnp.float32),
        pltpu.VMEM((L,), jnp.float32),
    ],
)


def kernel(pairs, k, u, beta):
    b = jnp.where(k == 0, jnp.float32(1.0), beta[k]).astype(jnp.float32)
    aux = jnp.concatenate([
        u,
        jnp.zeros((M_PAD - u.shape[0],), jnp.float32),
        jnp.full((L,), b, jnp.float32),
    ])
    partials = _sc_call(pairs.T, aux)
    return jnp.sum(partials)


# speedup vs baseline: 1.4780x; 1.0184x over previous
import functools

import jax
import jax.numpy as jnp
from jax import lax
from jax.experimental import pallas as pl
from jax.experimental.pallas import tpu as pltpu
from jax.experimental.pallas import tpu_sc as plsc

N_PAIRS = 16384
M_PAD = 1024
K_PAD = 32
L = 16
NW = 16
PAIRS_PER_W = N_PAIRS // NW
CHUNKS = PAIRS_PER_W // L
AUX = M_PAD + K_PAD + L

_LN2 = 0.6931471805599453
_SQRT2 = 1.4142135


def _ln(y):
    yi = lax.bitcast_convert_type(y, jnp.int32)
    e = (yi >> 23) - 127
    m = lax.bitcast_convert_type((yi & 0x7FFFFF) | 0x3F800000, jnp.float32)
    big = m > _SQRT2
    m = jnp.where(big, m * 0.5, m)
    ef = (e + big.astype(jnp.int32)).astype(jnp.float32)
    t = (m - 1.0) / (m + 1.0)
    t2 = t * t
    p = 2.0 * t * (1.0 + t2 * (1.0 / 3.0 + t2 * (0.2 + t2 * (1.0 / 7.0))))
    return ef * _LN2 + p


def _body(pairs_hbm, aux_hbm, out_hbm, pairs_v, aux_v, acc_v):
    wid = lax.axis_index("s")
    col0 = wid * PAIRS_PER_W
    pltpu.sync_copy(pairs_hbm.at[:, pl.ds(col0, PAIRS_PER_W)], pairs_v)
    pltpu.sync_copy(aux_hbm, aux_v)

    ki = aux_v[pl.ds(M_PAD + K_PAD, L)].astype(jnp.int32)
    bb = plsc.load_gather(aux_v, [ki + M_PAD])
    b = jnp.where(ki == 0, 1.0, bb)

    def chunk(j, acc):
        w_idx = pairs_v[0, pl.ds(j * L, L)]
        l_idx = pairs_v[1, pl.ds(j * L, L)]
        uw = plsc.load_gather(aux_v, [w_idx])
        ul = plsc.load_gather(aux_v, [l_idx])
        y = 1.0 + jnp.exp(b * (ul - uw))
        return acc + _ln(y)

    acc_v[...] = lax.fori_loop(0, CHUNKS, chunk, jnp.zeros((L,), jnp.float32))
    pltpu.sync_copy(acc_v, out_hbm.at[wid])


_sc_call = pl.kernel(
    _body,
    out_type=jax.ShapeDtypeStruct((NW, L), jnp.float32),
    mesh=plsc.VectorSubcoreMesh(
        core_axis_name="c", subcore_axis_name="s", num_cores=1, num_subcores=16),
    compiler_params=pltpu.CompilerParams(needs_layout_passes=False),
    scratch_types=[
        pltpu.VMEM((2, PAIRS_PER_W), jnp.int32),
        pltpu.VMEM((AUX,), jnp.float32),
        pltpu.VMEM((L,), jnp.float32),
    ],
)


def kernel(pairs, k, u, beta):
    aux = jnp.concatenate([
        u,
        jnp.zeros((M_PAD - u.shape[0],), jnp.float32),
        beta.astype(jnp.float32),
        jnp.zeros((K_PAD - beta.shape[0],), jnp.float32),
        jnp.full((L,), k, jnp.float32),
    ])
    partials = _sc_call(pairs.T, aux)
    return jnp.sum(partials)
